# baseline (speedup 1.0000x reference)
import functools

import jax
import jax.numpy as jnp
from jax.experimental import pallas as pl
from jax.experimental.pallas import tpu as pltpu

_N_HEADS = 16
_F32 = jnp.float32


def _dot(a, b):
    return jnp.dot(a, b, preferred_element_type=_F32)


def _dot_nt(a, b):
    return jax.lax.dot_general(a, b, (((1,), (1,)), ((), ())),
                               preferred_element_type=_F32)


def _ln_route_body(x_ref, imp_ref, w_ref, g_ref, b_ref,
                   h1_ref, wacc_ref, isum_ref, *, n_groups):
    s = pl.program_id(1)
    xb = x_ref[0]
    mu = jnp.mean(xb, axis=-1, keepdims=True)
    xc = xb - mu
    var = jnp.mean(xc * xc, axis=-1, keepdims=True)
    h1 = xc * jax.lax.rsqrt(var + 1e-5) * g_ref[...] + b_ref[...]
    h1_ref[0] = h1
    logits = _dot(h1, w_ref[...])
    ps = []
    for r in range(n_groups):
        lg = logits[:, r * 64:(r + 1) * 64]
        m = jnp.max(lg, axis=-1, keepdims=True)
        e = jnp.exp(lg - m)
        ps.append(e / jnp.sum(e, axis=-1, keepdims=True))
    p = jnp.concatenate(ps, axis=-1) if n_groups > 1 else ps[0]
    imp = imp_ref[0, 0]

    @pl.when(s == 0)
    def _():
        wacc_ref[...] = jnp.zeros_like(wacc_ref)
        isum_ref[...] = jnp.zeros_like(isum_ref)

    wacc_ref[0] += _dot(imp, p)
    isum_ref[0] += jnp.sum(imp)


def _ln_route(x, importance, w_cat, g, b, n_groups, bs):
    bsz, ssz, d = x.shape
    grid = (bsz, ssz // bs)
    imp4 = importance.reshape(bsz, ssz // bs, 1, bs)
    h1, wacc, isum = pl.pallas_call(
        functools.partial(_ln_route_body, n_groups=n_groups),
        grid=grid,
        in_specs=[
            pl.BlockSpec((1, bs, d), lambda bi, si: (bi, si, 0)),
            pl.BlockSpec((1, 1, 1, bs), lambda bi, si: (bi, si, 0, 0)),
            pl.BlockSpec((d, n_groups * 64), lambda bi, si: (0, 0)),
            pl.BlockSpec((1, d), lambda bi, si: (0, 0)),
            pl.BlockSpec((1, d), lambda bi, si: (0, 0)),
        ],
        out_specs=[
            pl.BlockSpec((1, bs, d), lambda bi, si: (bi, si, 0)),
            pl.BlockSpec((1, 1, n_groups * 64), lambda bi, si: (bi, 0, 0)),
            pl.BlockSpec((1, 1, 128), lambda bi, si: (bi, 0, 0)),
        ],
        out_shape=[
            jax.ShapeDtypeStruct((bsz, ssz, d), _F32),
            jax.ShapeDtypeStruct((bsz, 1, n_groups * 64), _F32),
            jax.ShapeDtypeStruct((bsz, 1, 128), _F32),
        ],
    )(x, imp4, w_cat, g.reshape(1, d), b.reshape(1, d))
    return h1, wacc.reshape(bsz, n_groups * 64), isum.reshape(bsz, 128)


def _topk_body(wacc_ref, isum_ref, tw_ref, ix_ref, *, ks):
    c = isum_ref[:, :1] + 1e-8
    tws, ixs = [], []
    for r, k in enumerate(ks):
        w = wacc_ref[:, r * 64:(r + 1) * 64] / c
        w = w / (jnp.sum(w, axis=-1, keepdims=True) + 1e-8)
        iota = jax.lax.broadcasted_iota(jnp.int32, w.shape, 1)
        vals = w
        sel_v, sel_i = [], []
        for _ in range(k):
            m = jnp.max(vals, axis=-1, keepdims=True)
            idx = jnp.min(jnp.where(vals == m, iota, 64), axis=-1,
                          keepdims=True)
            oh = iota == idx
            sel_v.append(m)
            sel_i.append(idx)
            vals = jnp.where(oh, -jnp.inf, vals)
        tw = jnp.concatenate(sel_v, axis=1)
        tw = tw / (jnp.sum(tw, axis=-1, keepdims=True) + 1e-8)
        ix = jnp.concatenate(sel_i, axis=1)
        if k < 16:
            pad_v = jnp.zeros_like(tw[:, :16 - k])
            tw = jnp.concatenate([tw, pad_v], axis=1)
            ix = jnp.concatenate([ix, pad_v.astype(jnp.int32)], axis=1)
        tws.append(tw)
        ixs.append(ix)
    tw_ref[...] = jnp.concatenate(tws, axis=-1) if len(ks) > 1 else tws[0]
    ix_ref[...] = jnp.concatenate(ixs, axis=-1) if len(ks) > 1 else ixs[0]


def _topk(wacc, isum, ks):
    bsz = wacc.shape[0]
    return pl.pallas_call(
        functools.partial(_topk_body, ks=ks),
        out_shape=[
            jax.ShapeDtypeStruct((bsz, 16 * len(ks)), _F32),
            jax.ShapeDtypeStruct((bsz, 16 * len(ks)), jnp.int32),
        ],
    )(wacc, isum)


def _gcomb_body(idx_ref, tw_ref, pool_ref, out_ref, *, k):
    del idx_ref
    row = pl.program_id(0)
    j = pl.program_id(2)

    @pl.when(j == 0)
    def _():
        out_ref[...] = jnp.zeros_like(out_ref)

    out_ref[...] += tw_ref[row * k + j] * pool_ref[...]


def _gcomb(idx, tw, pool, rows, k, bd):
    n, dim1, dim2 = pool.shape
    nch = dim1 // bd
    return pl.pallas_call(
        functools.partial(_gcomb_body, k=k),
        grid_spec=pltpu.PrefetchScalarGridSpec(
            num_scalar_prefetch=2,
            grid=(rows, nch, k),
            in_specs=[
                pl.BlockSpec((1, bd, dim2),
                             lambda r, c, j, idx_ref, tw_ref:
                             (idx_ref[r * k + j], c, 0)),
            ],
            out_specs=pl.BlockSpec((1, bd, dim2),
                                   lambda r, c, j, idx_ref, tw_ref:
                                   (r, c, 0)),
        ),
        out_shape=jax.ShapeDtypeStruct((rows, dim1, dim2), _F32),
    )(idx, tw, pool)


def _qkv_body(h1_ref, sc_ref, sq_ref, sk_ref, sv_ref, q_ref, k_ref, v_ref):
    h = _dot(h1_ref[0], sc_ref[0])
    q_ref[0] = _dot(h, sq_ref[0])
    k_ref[0] = _dot(h, sk_ref[0])
    v_ref[0] = _dot(h, sv_ref[0])


def _qkv(h1, shared_c, sq, sk, sv, bs):
    bsz, ssz, d = h1.shape
    r = shared_c.shape[-1]
    big = pl.BlockSpec((1, bs, d), lambda bi, si: (bi, si, 0))
    mat_dr = pl.BlockSpec((1, d, r), lambda bi, si: (bi, 0, 0))
    mat_rd = pl.BlockSpec((1, r, d), lambda bi, si: (bi, 0, 0))
    out = jax.ShapeDtypeStruct((bsz, ssz, d), _F32)
    return pl.pallas_call(
        _qkv_body,
        grid=(bsz, ssz // bs),
        in_specs=[big, mat_dr, mat_rd, mat_rd, mat_rd],
        out_specs=[big, big, big],
        out_shape=[out, out, out],
    )(h1, shared_c, sq, sk, sv)


def _attn_body(q_ref, k_ref, v_ref, x_ref, wo_ref, out_ref, *, scale,
               n_heads):
    q = q_ref[0]
    k = k_ref[0]
    v = v_ref[0]
    dh = q.shape[-1] // n_heads
    outs = []
    for hh in range(n_heads):
        sl = slice(hh * dh, (hh + 1) * dh)
        s = _dot_nt(q[:, sl], k[:, sl]) * scale
        m = jnp.max(s, axis=-1, keepdims=True)
        p = jnp.exp(s - m)
        p = p / jnp.sum(p, axis=-1, keepdims=True)
        outs.append(_dot(p, v[:, sl]))
    o = jnp.concatenate(outs, axis=-1)
    out_ref[0] = x_ref[0] + _dot(o, wo_ref[...])


def _attn(q, k, v, x, w_o, bq, scale, n_heads):
    bsz, ssz, d = q.shape
    blk = pl.BlockSpec((1, bq, d), lambda bi, qi: (bi, qi, 0))
    full = pl.BlockSpec((1, ssz, d), lambda bi, qi: (bi, 0, 0))
    return pl.pallas_call(
        functools.partial(_attn_body, scale=scale, n_heads=n_heads),
        grid=(bsz, ssz // bq),
        in_specs=[blk, full, full, blk,
                  pl.BlockSpec((d, d), lambda bi, qi: (0, 0))],
        out_specs=blk,
        out_shape=jax.ShapeDtypeStruct((bsz, ssz, d), _F32),
    )(q, k, v, x, w_o)


def _mem_body(h2_ref, sm_ref, kk_ref, kv_ref, x1_ref, out_ref, *, scale):
    hm = _dot(h2_ref[0], sm_ref[0])
    s = _dot_nt(hm, kk_ref[...]) * scale
    m = jnp.max(s, axis=-1, keepdims=True)
    p = jnp.exp(s - m)
    p = p / jnp.sum(p, axis=-1, keepdims=True)
    out_ref[0] = x1_ref[0] + _dot(p, kv_ref[...])


def _mem(h2, shared_m, k_k, k_v, x1, bs, scale):
    bsz, ssz, d = h2.shape
    r = shared_m.shape[-1]
    nk = k_k.shape[0]
    big = pl.BlockSpec((1, bs, d), lambda bi, si: (bi, si, 0))
    return pl.pallas_call(
        functools.partial(_mem_body, scale=scale),
        grid=(bsz, ssz // bs),
        in_specs=[
            big,
            pl.BlockSpec((1, d, r), lambda bi, si: (bi, 0, 0)),
            pl.BlockSpec((nk, r), lambda bi, si: (0, 0)),
            pl.BlockSpec((nk, d), lambda bi, si: (0, 0)),
            big,
        ],
        out_specs=big,
        out_shape=jax.ShapeDtypeStruct((bsz, ssz, d), _F32),
    )(h2, shared_m, k_k, k_v, x1)


def kernel(x, importance, W_compress_router, W_expand_router_Q,
           W_expand_router_K, W_expand_router_V, W_memory_router,
           compress_neurons, expand_neurons_pool, knowledge_K, knowledge_V,
           W_O, g1, b1, g2, b2):
    bsz, ssz, d = x.shape
    n_c, _, r = compress_neurons.shape
    nk = knowledge_K.shape[0]
    h = _N_HEADS
    dh = d // h
    bs = min(512, ssz)

    w_cat = jnp.concatenate([W_compress_router, W_expand_router_Q,
                             W_expand_router_K, W_expand_router_V], axis=1)
    h1, wacc, isum = _ln_route(x, importance, w_cat, g1, b1, 4, bs)
    tw, ix = _topk(wacc, isum, (16, 8, 8, 8))

    c_idx = ix[:, :16].reshape(bsz * 16)
    c_tw = tw[:, :16].reshape(bsz * 16)
    e_idx = ix[:, 16:].reshape(bsz, 3, 16)[:, :, :8].transpose(
        1, 0, 2).reshape(3 * bsz * 8)
    e_tw = tw[:, 16:].reshape(bsz, 3, 16)[:, :, :8].transpose(
        1, 0, 2).reshape(3 * bsz * 8)
    shared_c = _gcomb(c_idx, c_tw, compress_neurons, bsz, 16, 256)
    shared_e = _gcomb(e_idx, e_tw, expand_neurons_pool, 3 * bsz, 8,
                      64).reshape(3, bsz, r, d)

    q, k, v = _qkv(h1, shared_c, shared_e[0], shared_e[1], shared_e[2], bs)
    x1 = _attn(q, k, v, x, W_O, bs, 1.0 / (dh ** 0.5), h)

    h2, wacc_m, isum_m = _ln_route(x1, importance, W_memory_router, g2, b2,
                                   1, bs)
    tw_m, ix_m = _topk(wacc_m, isum_m, (16,))
    shared_m = _gcomb(ix_m.reshape(bsz * 16), tw_m.reshape(bsz * 16),
                      compress_neurons, bsz, 16, 256)
    mbs = min(256, ssz)
    return _mem(h2, shared_m, knowledge_K, knowledge_V, x1, mbs,
                1.0 / (r ** 0.5))


# speedup vs baseline: 1.2733x; 1.2733x over previous
import functools

import jax
import jax.numpy as jnp
from jax.experimental import pallas as pl
from jax.experimental.pallas import tpu as pltpu

_N_HEADS = 16
_F32 = jnp.float32


def _dot(a, b):
    return jnp.dot(a, b, preferred_element_type=_F32)


def _dot_nt(a, b):
    return jax.lax.dot_general(a, b, (((1,), (1,)), ((), ())),
                               preferred_element_type=_F32)


def _ln_route_body(x_ref, imp_ref, w_ref, g_ref, b_ref,
                   h1_ref, wacc_ref, isum_ref, *, n_groups):
    s = pl.program_id(1)
    xb = x_ref[0]
    mu = jnp.mean(xb, axis=-1, keepdims=True)
    xc = xb - mu
    var = jnp.mean(xc * xc, axis=-1, keepdims=True)
    h1 = xc * jax.lax.rsqrt(var + 1e-5) * g_ref[...] + b_ref[...]
    h1_ref[0] = h1
    logits = _dot(h1, w_ref[...])
    ps = []
    for r in range(n_groups):
        lg = logits[:, r * 64:(r + 1) * 64]
        m = jnp.max(lg, axis=-1, keepdims=True)
        e = jnp.exp(lg - m)
        ps.append(e / jnp.sum(e, axis=-1, keepdims=True))
    p = jnp.concatenate(ps, axis=-1) if n_groups > 1 else ps[0]
    imp = imp_ref[0, 0]

    @pl.when(s == 0)
    def _():
        wacc_ref[...] = jnp.zeros_like(wacc_ref)
        isum_ref[...] = jnp.zeros_like(isum_ref)

    wacc_ref[0] += _dot(imp, p)
    isum_ref[0] += jnp.sum(imp)


def _ln_route(x, importance, w_cat, g, b, n_groups, bs):
    bsz, ssz, d = x.shape
    grid = (bsz, ssz // bs)
    imp4 = importance.reshape(bsz, ssz // bs, 1, bs)
    h1, wacc, isum = pl.pallas_call(
        functools.partial(_ln_route_body, n_groups=n_groups),
        grid=grid,
        in_specs=[
            pl.BlockSpec((1, bs, d), lambda bi, si: (bi, si, 0)),
            pl.BlockSpec((1, 1, 1, bs), lambda bi, si: (bi, si, 0, 0)),
            pl.BlockSpec((d, n_groups * 64), lambda bi, si: (0, 0)),
            pl.BlockSpec((1, d), lambda bi, si: (0, 0)),
            pl.BlockSpec((1, d), lambda bi, si: (0, 0)),
        ],
        out_specs=[
            pl.BlockSpec((1, bs, d), lambda bi, si: (bi, si, 0)),
            pl.BlockSpec((1, 1, n_groups * 64), lambda bi, si: (bi, 0, 0)),
            pl.BlockSpec((1, 1, 128), lambda bi, si: (bi, 0, 0)),
        ],
        out_shape=[
            jax.ShapeDtypeStruct((bsz, ssz, d), _F32),
            jax.ShapeDtypeStruct((bsz, 1, n_groups * 64), _F32),
            jax.ShapeDtypeStruct((bsz, 1, 128), _F32),
        ],
    )(x, imp4, w_cat, g.reshape(1, d), b.reshape(1, d))
    return h1, wacc.reshape(bsz, n_groups * 64), isum.reshape(bsz, 128)


def _topk_body(wacc_ref, isum_ref, tw_ref, ix_ref, *, ks):
    c = isum_ref[:, :1] + 1e-8
    tws, ixs = [], []
    for r, k in enumerate(ks):
        w = wacc_ref[:, r * 64:(r + 1) * 64] / c
        w = w / (jnp.sum(w, axis=-1, keepdims=True) + 1e-8)
        iota = jax.lax.broadcasted_iota(jnp.int32, w.shape, 1)
        vals = w
        sel_v, sel_i = [], []
        for _ in range(k):
            m = jnp.max(vals, axis=-1, keepdims=True)
            idx = jnp.min(jnp.where(vals == m, iota, 64), axis=-1,
                          keepdims=True)
            oh = iota == idx
            sel_v.append(m)
            sel_i.append(idx)
            vals = jnp.where(oh, -jnp.inf, vals)
        tw = jnp.concatenate(sel_v, axis=1)
        tw = tw / (jnp.sum(tw, axis=-1, keepdims=True) + 1e-8)
        ix = jnp.concatenate(sel_i, axis=1)
        if k < 16:
            pad_v = jnp.zeros_like(tw[:, :16 - k])
            tw = jnp.concatenate([tw, pad_v], axis=1)
            ix = jnp.concatenate([ix, pad_v.astype(jnp.int32)], axis=1)
        tws.append(tw)
        ixs.append(ix)
    tw_ref[...] = jnp.concatenate(tws, axis=-1) if len(ks) > 1 else tws[0]
    ix_ref[...] = jnp.concatenate(ixs, axis=-1) if len(ks) > 1 else ixs[0]


def _topk(wacc, isum, ks):
    bsz = wacc.shape[0]
    return pl.pallas_call(
        functools.partial(_topk_body, ks=ks),
        out_shape=[
            jax.ShapeDtypeStruct((bsz, 16 * len(ks)), _F32),
            jax.ShapeDtypeStruct((bsz, 16 * len(ks)), jnp.int32),
        ],
    )(wacc, isum)


def _gcomb_body(idx_ref, tw_ref, pool_ref, out_ref, *, k):
    del idx_ref
    row = pl.program_id(0)
    j = pl.program_id(2)

    @pl.when(j == 0)
    def _():
        out_ref[...] = jnp.zeros_like(out_ref)

    out_ref[...] += tw_ref[row * k + j] * pool_ref[...]


def _gcomb(idx, tw, pool, rows, k, bd):
    n, dim1, dim2 = pool.shape
    nch = dim1 // bd
    return pl.pallas_call(
        functools.partial(_gcomb_body, k=k),
        grid_spec=pltpu.PrefetchScalarGridSpec(
            num_scalar_prefetch=2,
            grid=(rows, nch, k),
            in_specs=[
                pl.BlockSpec((1, bd, dim2),
                             lambda r, c, j, idx_ref, tw_ref:
                             (idx_ref[r * k + j], c, 0)),
            ],
            out_specs=pl.BlockSpec((1, bd, dim2),
                                   lambda r, c, j, idx_ref, tw_ref:
                                   (r, c, 0)),
        ),
        out_shape=jax.ShapeDtypeStruct((rows, dim1, dim2), _F32),
    )(idx, tw, pool)


def _qkv_body(h1_ref, sc_ref, sq_ref, sk_ref, sv_ref, q_ref, k_ref, v_ref):
    h = _dot(h1_ref[0], sc_ref[0])
    q_ref[0] = _dot(h, sq_ref[0])
    k_ref[0] = _dot(h, sk_ref[0])
    v_ref[0] = _dot(h, sv_ref[0])


def _qkv(h1, shared_c, sq, sk, sv, bs):
    bsz, ssz, d = h1.shape
    r = shared_c.shape[-1]
    big = pl.BlockSpec((1, bs, d), lambda bi, si: (bi, si, 0))
    mat_dr = pl.BlockSpec((1, d, r), lambda bi, si: (bi, 0, 0))
    mat_rd = pl.BlockSpec((1, r, d), lambda bi, si: (bi, 0, 0))
    out = jax.ShapeDtypeStruct((bsz, ssz, d), _F32)
    return pl.pallas_call(
        _qkv_body,
        grid=(bsz, ssz // bs),
        in_specs=[big, mat_dr, mat_rd, mat_rd, mat_rd],
        out_specs=[big, big, big],
        out_shape=[out, out, out],
    )(h1, shared_c, sq, sk, sv)


def _attn_body(q_ref, k_ref, v_ref, x_ref, wo_ref, out_ref, *, scale,
               n_heads):
    q = q_ref[0]
    k = k_ref[0]
    v = v_ref[0]
    dh = q.shape[-1] // n_heads
    outs = []
    for hh in range(n_heads):
        sl = slice(hh * dh, (hh + 1) * dh)
        s = _dot_nt(q[:, sl], k[:, sl]) * scale
        m = jnp.max(s, axis=-1, keepdims=True)
        p = jnp.exp(s - m)
        p = p / jnp.sum(p, axis=-1, keepdims=True)
        outs.append(_dot(p, v[:, sl]))
    o = jnp.concatenate(outs, axis=-1)
    out_ref[0] = x_ref[0] + _dot(o, wo_ref[...])


def _attn(q, k, v, x, w_o, bq, scale, n_heads):
    bsz, ssz, d = q.shape
    blk = pl.BlockSpec((1, bq, d), lambda bi, qi: (bi, qi, 0))
    full = pl.BlockSpec((1, ssz, d), lambda bi, qi: (bi, 0, 0))
    return pl.pallas_call(
        functools.partial(_attn_body, scale=scale, n_heads=n_heads),
        grid=(bsz, ssz // bq),
        in_specs=[blk, full, full, blk,
                  pl.BlockSpec((d, d), lambda bi, qi: (0, 0))],
        out_specs=blk,
        out_shape=jax.ShapeDtypeStruct((bsz, ssz, d), _F32),
    )(q, k, v, x, w_o)


def _mem_body(h2_ref, sm_ref, kk_ref, kv_ref, x1_ref, out_ref, *, scale):
    hm = _dot(h2_ref[0], sm_ref[0])
    s = _dot_nt(hm, kk_ref[...]) * scale
    m = jnp.max(s, axis=-1, keepdims=True)
    p = jnp.exp(s - m)
    p = p / jnp.sum(p, axis=-1, keepdims=True)
    out_ref[0] = x1_ref[0] + _dot(p, kv_ref[...])


def _mem(h2, shared_m, k_k, k_v, x1, bs, scale):
    bsz, ssz, d = h2.shape
    r = shared_m.shape[-1]
    nk = k_k.shape[0]
    big = pl.BlockSpec((1, bs, d), lambda bi, si: (bi, si, 0))
    return pl.pallas_call(
        functools.partial(_mem_body, scale=scale),
        grid=(bsz, ssz // bs),
        in_specs=[
            big,
            pl.BlockSpec((1, d, r), lambda bi, si: (bi, 0, 0)),
            pl.BlockSpec((nk, r), lambda bi, si: (0, 0)),
            pl.BlockSpec((nk, d), lambda bi, si: (0, 0)),
            big,
        ],
        out_specs=big,
        out_shape=jax.ShapeDtypeStruct((bsz, ssz, d), _F32),
    )(h2, shared_m, k_k, k_v, x1)


def kernel(x, importance, W_compress_router, W_expand_router_Q,
           W_expand_router_K, W_expand_router_V, W_memory_router,
           compress_neurons, expand_neurons_pool, knowledge_K, knowledge_V,
           W_O, g1, b1, g2, b2):
    bsz, ssz, d = x.shape
    n_c, _, r = compress_neurons.shape
    nk = knowledge_K.shape[0]
    h = _N_HEADS
    dh = d // h
    bs = min(512, ssz)

    w_cat = jnp.concatenate([W_compress_router, W_expand_router_Q,
                             W_expand_router_K, W_expand_router_V], axis=1)
    h1, wacc, isum = _ln_route(x, importance, w_cat, g1, b1, 4, bs)
    tw, ix = _topk(wacc, isum, (16, 8, 8, 8))

    c_idx = ix[:, :16].reshape(bsz * 16)
    c_tw = tw[:, :16].reshape(bsz * 16)
    e_idx = ix[:, 16:].reshape(bsz, 3, 16)[:, :, :8].transpose(
        1, 0, 2).reshape(3 * bsz * 8)
    e_tw = tw[:, 16:].reshape(bsz, 3, 16)[:, :, :8].transpose(
        1, 0, 2).reshape(3 * bsz * 8)
    shared_c = _gcomb(c_idx, c_tw, compress_neurons, bsz, 16, d)
    shared_e = _gcomb(e_idx, e_tw, expand_neurons_pool, 3 * bsz, 8,
                      r).reshape(3, bsz, r, d)

    q, k, v = _qkv(h1, shared_c, shared_e[0], shared_e[1], shared_e[2], bs)
    x1 = _attn(q, k, v, x, W_O, bs, 1.0 / (dh ** 0.5), h)

    h2, wacc_m, isum_m = _ln_route(x1, importance, W_memory_router, g2, b2,
                                   1, bs)
    tw_m, ix_m = _topk(wacc_m, isum_m, (16,))
    shared_m = _gcomb(ix_m.reshape(bsz * 16), tw_m.reshape(bsz * 16),
                      compress_neurons, bsz, 16, d)
    mbs = min(256, ssz)
    return _mem(h2, shared_m, knowledge_K, knowledge_V, x1, mbs,
                1.0 / (r ** 0.5))


# speedup vs baseline: 1.2736x; 1.0003x over previous
import functools

import jax
import jax.numpy as jnp
from jax.experimental import pallas as pl
from jax.experimental.pallas import tpu as pltpu

_N_HEADS = 16
_F32 = jnp.float32
_BF16 = jnp.bfloat16


def _dot(a, b):
    return jnp.dot(a, b, preferred_element_type=_F32)


def _dot_nt(a, b):
    return jax.lax.dot_general(a, b, (((1,), (1,)), ((), ())),
                               preferred_element_type=_F32)


def _ln_route_body(x_ref, imp_ref, w_ref, g_ref, b_ref,
                   h1_ref, wacc_ref, isum_ref, *, n_groups):
    s = pl.program_id(1)
    xb = x_ref[0]
    mu = jnp.mean(xb, axis=-1, keepdims=True)
    xc = xb - mu
    var = jnp.mean(xc * xc, axis=-1, keepdims=True)
    h1 = xc * jax.lax.rsqrt(var + 1e-5) * g_ref[...] + b_ref[...]
    h1_ref[0] = h1.astype(jnp.bfloat16)
    logits = _dot(h1, w_ref[...])
    ps = []
    for r in range(n_groups):
        lg = logits[:, r * 64:(r + 1) * 64]
        m = jnp.max(lg, axis=-1, keepdims=True)
        e = jnp.exp(lg - m)
        ps.append(e / jnp.sum(e, axis=-1, keepdims=True))
    p = jnp.concatenate(ps, axis=-1) if n_groups > 1 else ps[0]
    imp = imp_ref[0, 0]

    @pl.when(s == 0)
    def _():
        wacc_ref[...] = jnp.zeros_like(wacc_ref)
        isum_ref[...] = jnp.zeros_like(isum_ref)

    wacc_ref[0] += _dot(imp, p)
    isum_ref[0] += jnp.sum(imp)


def _ln_route(x, importance, w_cat, g, b, n_groups, bs):
    bsz, ssz, d = x.shape
    grid = (bsz, ssz // bs)
    imp4 = importance.reshape(bsz, ssz // bs, 1, bs)
    h1, wacc, isum = pl.pallas_call(
        functools.partial(_ln_route_body, n_groups=n_groups),
        grid=grid,
        in_specs=[
            pl.BlockSpec((1, bs, d), lambda bi, si: (bi, si, 0)),
            pl.BlockSpec((1, 1, 1, bs), lambda bi, si: (bi, si, 0, 0)),
            pl.BlockSpec((d, n_groups * 64), lambda bi, si: (0, 0)),
            pl.BlockSpec((1, d), lambda bi, si: (0, 0)),
            pl.BlockSpec((1, d), lambda bi, si: (0, 0)),
        ],
        out_specs=[
            pl.BlockSpec((1, bs, d), lambda bi, si: (bi, si, 0)),
            pl.BlockSpec((1, 1, n_groups * 64), lambda bi, si: (bi, 0, 0)),
            pl.BlockSpec((1, 1, 128), lambda bi, si: (bi, 0, 0)),
        ],
        out_shape=[
            jax.ShapeDtypeStruct((bsz, ssz, d), _BF16),
            jax.ShapeDtypeStruct((bsz, 1, n_groups * 64), _F32),
            jax.ShapeDtypeStruct((bsz, 1, 128), _F32),
        ],
    )(x, imp4, w_cat, g.reshape(1, d), b.reshape(1, d))
    return h1, wacc.reshape(bsz, n_groups * 64), isum.reshape(bsz, 128)


def _topk_body(wacc_ref, isum_ref, tw_ref, ix_ref, *, ks):
    c = isum_ref[:, :1] + 1e-8
    tws, ixs = [], []
    for r, k in enumerate(ks):
        w = wacc_ref[:, r * 64:(r + 1) * 64] / c
        w = w / (jnp.sum(w, axis=-1, keepdims=True) + 1e-8)
        iota = jax.lax.broadcasted_iota(jnp.int32, w.shape, 1)
        vals = w
        sel_v, sel_i = [], []
        for _ in range(k):
            m = jnp.max(vals, axis=-1, keepdims=True)
            idx = jnp.min(jnp.where(vals == m, iota, 64), axis=-1,
                          keepdims=True)
            oh = iota == idx
            sel_v.append(m)
            sel_i.append(idx)
            vals = jnp.where(oh, -jnp.inf, vals)
        tw = jnp.concatenate(sel_v, axis=1)
        tw = tw / (jnp.sum(tw, axis=-1, keepdims=True) + 1e-8)
        ix = jnp.concatenate(sel_i, axis=1)
        if k < 16:
            pad_v = jnp.zeros_like(tw[:, :16 - k])
            tw = jnp.concatenate([tw, pad_v], axis=1)
            ix = jnp.concatenate([ix, pad_v.astype(jnp.int32)], axis=1)
        tws.append(tw)
        ixs.append(ix)
    tw_ref[...] = jnp.concatenate(tws, axis=-1) if len(ks) > 1 else tws[0]
    ix_ref[...] = jnp.concatenate(ixs, axis=-1) if len(ks) > 1 else ixs[0]


def _topk(wacc, isum, ks):
    bsz = wacc.shape[0]
    return pl.pallas_call(
        functools.partial(_topk_body, ks=ks),
        out_shape=[
            jax.ShapeDtypeStruct((bsz, 16 * len(ks)), _F32),
            jax.ShapeDtypeStruct((bsz, 16 * len(ks)), jnp.int32),
        ],
    )(wacc, isum)


def _gcomb_body(idx_ref, tw_ref, pool_ref, out_ref, *, k):
    del idx_ref
    row = pl.program_id(0)
    j = pl.program_id(2)

    @pl.when(j == 0)
    def _():
        out_ref[...] = jnp.zeros_like(out_ref)

    out_ref[...] += tw_ref[row * k + j] * pool_ref[...]


def _gcomb(idx, tw, pool, rows, k, bd):
    n, dim1, dim2 = pool.shape
    nch = dim1 // bd
    return pl.pallas_call(
        functools.partial(_gcomb_body, k=k),
        grid_spec=pltpu.PrefetchScalarGridSpec(
            num_scalar_prefetch=2,
            grid=(rows, nch, k),
            in_specs=[
                pl.BlockSpec((1, bd, dim2),
                             lambda r, c, j, idx_ref, tw_ref:
                             (idx_ref[r * k + j], c, 0)),
            ],
            out_specs=pl.BlockSpec((1, bd, dim2),
                                   lambda r, c, j, idx_ref, tw_ref:
                                   (r, c, 0)),
        ),
        out_shape=jax.ShapeDtypeStruct((rows, dim1, dim2), _F32),
    )(idx, tw, pool)


def _qkv_body(h1_ref, sc_ref, sq_ref, sk_ref, sv_ref, q_ref, k_ref, v_ref):
    h = _dot(h1_ref[0], sc_ref[0]).astype(jnp.bfloat16)
    q_ref[0] = _dot(h, sq_ref[0]).astype(jnp.bfloat16)
    k_ref[0] = _dot(h, sk_ref[0]).astype(jnp.bfloat16)
    v_ref[0] = _dot(h, sv_ref[0]).astype(jnp.bfloat16)


def _qkv(h1, shared_c, sq, sk, sv, bs):
    bsz, ssz, d = h1.shape
    r = shared_c.shape[-1]
    big = pl.BlockSpec((1, bs, d), lambda bi, si: (bi, si, 0))
    mat_dr = pl.BlockSpec((1, d, r), lambda bi, si: (bi, 0, 0))
    mat_rd = pl.BlockSpec((1, r, d), lambda bi, si: (bi, 0, 0))
    out = jax.ShapeDtypeStruct((bsz, ssz, d), _BF16)
    return pl.pallas_call(
        _qkv_body,
        grid=(bsz, ssz // bs),
        in_specs=[big, mat_dr, mat_rd, mat_rd, mat_rd],
        out_specs=[big, big, big],
        out_shape=[out, out, out],
    )(h1, shared_c, sq, sk, sv)


def _attn_body(q_ref, k_ref, v_ref, x_ref, wo_ref, out_ref, *, scale,
               n_heads):
    q = q_ref[0]
    k = k_ref[0]
    v = v_ref[0]
    dh = q.shape[-1] // n_heads
    outs = []
    for hh in range(n_heads):
        sl = slice(hh * dh, (hh + 1) * dh)
        s = _dot_nt(q[:, sl], k[:, sl]) * scale
        m = jnp.max(s, axis=-1, keepdims=True)
        p = jnp.exp(s - m)
        p = p / jnp.sum(p, axis=-1, keepdims=True)
        outs.append(_dot(p.astype(jnp.bfloat16), v[:, sl]))
    o = jnp.concatenate(outs, axis=-1).astype(jnp.bfloat16)
    out_ref[0] = x_ref[0] + _dot(o, wo_ref[...])


def _attn(q, k, v, x, w_o, bq, scale, n_heads):
    bsz, ssz, d = q.shape
    blk = pl.BlockSpec((1, bq, d), lambda bi, qi: (bi, qi, 0))
    full = pl.BlockSpec((1, ssz, d), lambda bi, qi: (bi, 0, 0))
    return pl.pallas_call(
        functools.partial(_attn_body, scale=scale, n_heads=n_heads),
        grid=(bsz, ssz // bq),
        in_specs=[blk, full, full, blk,
                  pl.BlockSpec((d, d), lambda bi, qi: (0, 0))],
        out_specs=blk,
        out_shape=jax.ShapeDtypeStruct((bsz, ssz, d), _F32),
    )(q, k, v, x, w_o)


def _mem_body(h2_ref, sm_ref, kk_ref, kv_ref, x1_ref, out_ref, *, scale):
    hm = _dot(h2_ref[0], sm_ref[0]).astype(jnp.bfloat16)
    s = _dot_nt(hm, kk_ref[...]) * scale
    m = jnp.max(s, axis=-1, keepdims=True)
    p = jnp.exp(s - m)
    p = p / jnp.sum(p, axis=-1, keepdims=True)
    out_ref[0] = x1_ref[0] + _dot(p.astype(jnp.bfloat16), kv_ref[...])


def _mem(h2, shared_m, k_k, k_v, x1, bs, scale):
    bsz, ssz, d = h2.shape
    r = shared_m.shape[-1]
    nk = k_k.shape[0]
    big = pl.BlockSpec((1, bs, d), lambda bi, si: (bi, si, 0))
    return pl.pallas_call(
        functools.partial(_mem_body, scale=scale),
        grid=(bsz, ssz // bs),
        in_specs=[
            big,
            pl.BlockSpec((1, d, r), lambda bi, si: (bi, 0, 0)),
            pl.BlockSpec((nk, r), lambda bi, si: (0, 0)),
            pl.BlockSpec((nk, d), lambda bi, si: (0, 0)),
            big,
        ],
        out_specs=big,
        out_shape=jax.ShapeDtypeStruct((bsz, ssz, d), _F32),
    )(h2, shared_m, k_k, k_v, x1)


def kernel(x, importance, W_compress_router, W_expand_router_Q,
           W_expand_router_K, W_expand_router_V, W_memory_router,
           compress_neurons, expand_neurons_pool, knowledge_K, knowledge_V,
           W_O, g1, b1, g2, b2):
    bsz, ssz, d = x.shape
    n_c, _, r = compress_neurons.shape
    nk = knowledge_K.shape[0]
    h = _N_HEADS
    dh = d // h
    bs = min(512, ssz)

    w_cat = jnp.concatenate([W_compress_router, W_expand_router_Q,
                             W_expand_router_K, W_expand_router_V], axis=1)
    h1, wacc, isum = _ln_route(x, importance, w_cat, g1, b1, 4, bs)
    tw, ix = _topk(wacc, isum, (16, 8, 8, 8))

    c_idx = ix[:, :16].reshape(bsz * 16)
    c_tw = tw[:, :16].reshape(bsz * 16)
    e_idx = ix[:, 16:].reshape(bsz, 3, 16)[:, :, :8].transpose(
        1, 0, 2).reshape(3 * bsz * 8)
    e_tw = tw[:, 16:].reshape(bsz, 3, 16)[:, :, :8].transpose(
        1, 0, 2).reshape(3 * bsz * 8)
    shared_c = _gcomb(c_idx, c_tw, compress_neurons, bsz, 16, d)
    shared_e = _gcomb(e_idx, e_tw, expand_neurons_pool, 3 * bsz, 8,
                      r).reshape(3, bsz, r, d)

    se16 = shared_e.astype(_BF16)
    q, k, v = _qkv(h1, shared_c.astype(_BF16), se16[0], se16[1], se16[2], bs)
    x1 = _attn(q, k, v, x, W_O.astype(_BF16), bs, 1.0 / (dh ** 0.5), h)

    h2, wacc_m, isum_m = _ln_route(x1, importance, W_memory_router, g2, b2,
                                   1, bs)
    tw_m, ix_m = _topk(wacc_m, isum_m, (16,))
    shared_m = _gcomb(ix_m.reshape(bsz * 16), tw_m.reshape(bsz * 16),
                      compress_neurons, bsz, 16, d)
    mbs = min(256, ssz)
    return _mem(h2, shared_m.astype(_BF16), knowledge_K.astype(_BF16),
                knowledge_V.astype(_BF16), x1, mbs, 1.0 / (r ** 0.5))


# speedup vs baseline: 1.6769x; 1.3166x over previous
import functools

import jax
import jax.numpy as jnp
from jax.experimental import pallas as pl
from jax.experimental.pallas import tpu as pltpu

_N_HEADS = 16
_F32 = jnp.float32
_BF16 = jnp.bfloat16


def _dot(a, b):
    return jnp.dot(a, b, preferred_element_type=_F32)


def _dot_nt(a, b):
    return jax.lax.dot_general(a, b, (((1,), (1,)), ((), ())),
                               preferred_element_type=_F32)


def _ln_route_body(x_ref, imp_ref, w_ref, g_ref, b_ref,
                   h1_ref, wacc_ref, isum_ref, *, n_groups):
    s = pl.program_id(1)
    xb = x_ref[0]
    mu = jnp.mean(xb, axis=-1, keepdims=True)
    xc = xb - mu
    var = jnp.mean(xc * xc, axis=-1, keepdims=True)
    h1 = xc * jax.lax.rsqrt(var + 1e-5) * g_ref[...] + b_ref[...]
    h1_ref[0] = h1.astype(jnp.bfloat16)
    logits = _dot(h1, w_ref[...])
    ps = []
    for r in range(n_groups):
        lg = logits[:, r * 64:(r + 1) * 64]
        m = jnp.max(lg, axis=-1, keepdims=True)
        e = jnp.exp(lg - m)
        ps.append(e / jnp.sum(e, axis=-1, keepdims=True))
    p = jnp.concatenate(ps, axis=-1) if n_groups > 1 else ps[0]
    imp = imp_ref[0, 0]

    @pl.when(s == 0)
    def _():
        wacc_ref[...] = jnp.zeros_like(wacc_ref)
        isum_ref[...] = jnp.zeros_like(isum_ref)

    wacc_ref[0] += _dot(imp, p)
    isum_ref[0] += jnp.sum(imp)


def _ln_route(x, importance, w_cat, g, b, n_groups, bs):
    bsz, ssz, d = x.shape
    grid = (bsz, ssz // bs)
    imp4 = importance.reshape(bsz, ssz // bs, 1, bs)
    h1, wacc, isum = pl.pallas_call(
        functools.partial(_ln_route_body, n_groups=n_groups),
        grid=grid,
        in_specs=[
            pl.BlockSpec((1, bs, d), lambda bi, si: (bi, si, 0)),
            pl.BlockSpec((1, 1, 1, bs), lambda bi, si: (bi, si, 0, 0)),
            pl.BlockSpec((d, n_groups * 64), lambda bi, si: (0, 0)),
            pl.BlockSpec((1, d), lambda bi, si: (0, 0)),
            pl.BlockSpec((1, d), lambda bi, si: (0, 0)),
        ],
        out_specs=[
            pl.BlockSpec((1, bs, d), lambda bi, si: (bi, si, 0)),
            pl.BlockSpec((1, 1, n_groups * 64), lambda bi, si: (bi, 0, 0)),
            pl.BlockSpec((1, 1, 128), lambda bi, si: (bi, 0, 0)),
        ],
        out_shape=[
            jax.ShapeDtypeStruct((bsz, ssz, d), _BF16),
            jax.ShapeDtypeStruct((bsz, 1, n_groups * 64), _F32),
            jax.ShapeDtypeStruct((bsz, 1, 128), _F32),
        ],
    )(x, imp4, w_cat, g.reshape(1, d), b.reshape(1, d))
    return h1, wacc.reshape(bsz, n_groups * 64), isum.reshape(bsz, 128)


def _topk_body(wacc_ref, isum_ref, tw_ref, ix_ref, *, ks):
    c = isum_ref[:, :1] + 1e-8
    tws, ixs = [], []
    for r, k in enumerate(ks):
        w = wacc_ref[:, r * 64:(r + 1) * 64] / c
        w = w / (jnp.sum(w, axis=-1, keepdims=True) + 1e-8)
        iota = jax.lax.broadcasted_iota(jnp.int32, w.shape, 1)
        vals = w
        sel_v, sel_i = [], []
        for _ in range(k):
            m = jnp.max(vals, axis=-1, keepdims=True)
            idx = jnp.min(jnp.where(vals == m, iota, 64), axis=-1,
                          keepdims=True)
            oh = iota == idx
            sel_v.append(m)
            sel_i.append(idx)
            vals = jnp.where(oh, -jnp.inf, vals)
        tw = jnp.concatenate(sel_v, axis=1)
        tw = tw / (jnp.sum(tw, axis=-1, keepdims=True) + 1e-8)
        ix = jnp.concatenate(sel_i, axis=1)
        if k < 16:
            pad_v = jnp.zeros_like(tw[:, :16 - k])
            tw = jnp.concatenate([tw, pad_v], axis=1)
            ix = jnp.concatenate([ix, pad_v.astype(jnp.int32)], axis=1)
        tws.append(tw)
        ixs.append(ix)
    tw_ref[...] = jnp.concatenate(tws, axis=-1) if len(ks) > 1 else tws[0]
    ix_ref[...] = jnp.concatenate(ixs, axis=-1) if len(ks) > 1 else ixs[0]


def _topk(wacc, isum, ks):
    bsz = wacc.shape[0]
    return pl.pallas_call(
        functools.partial(_topk_body, ks=ks),
        out_shape=[
            jax.ShapeDtypeStruct((bsz, 16 * len(ks)), _F32),
            jax.ShapeDtypeStruct((bsz, 16 * len(ks)), jnp.int32),
        ],
    )(wacc, isum)


def _gcomb_body(idx_ref, tw_ref, pool_ref, out_ref, *, k):
    del idx_ref
    row = pl.program_id(0)
    j = pl.program_id(2)

    @pl.when(j == 0)
    def _():
        out_ref[...] = jnp.zeros_like(out_ref)

    out_ref[...] += tw_ref[row * k + j] * pool_ref[...]


def _gcomb(idx, tw, pool, rows, k, bd):
    n, dim1, dim2 = pool.shape
    nch = dim1 // bd
    return pl.pallas_call(
        functools.partial(_gcomb_body, k=k),
        grid_spec=pltpu.PrefetchScalarGridSpec(
            num_scalar_prefetch=2,
            grid=(rows, nch, k),
            in_specs=[
                pl.BlockSpec((1, bd, dim2),
                             lambda r, c, j, idx_ref, tw_ref:
                             (idx_ref[r * k + j], c, 0)),
            ],
            out_specs=pl.BlockSpec((1, bd, dim2),
                                   lambda r, c, j, idx_ref, tw_ref:
                                   (r, c, 0)),
        ),
        out_shape=jax.ShapeDtypeStruct((rows, dim1, dim2), _F32),
    )(idx, tw, pool)


def _qkv_body(h1_ref, sc_ref, sq_ref, sk_ref, sv_ref, q_ref, k_ref, v_ref,
              *, qscale):
    h = _dot(h1_ref[0], sc_ref[0]).astype(jnp.bfloat16)
    q_ref[0] = (_dot(h, sq_ref[0]) * qscale).astype(jnp.bfloat16)
    k_ref[0] = _dot(h, sk_ref[0]).astype(jnp.bfloat16)
    v_ref[0] = _dot(h, sv_ref[0]).astype(jnp.bfloat16)


def _qkv(h1, shared_c, sq, sk, sv, bs, qscale):
    bsz, ssz, d = h1.shape
    r = shared_c.shape[-1]
    big = pl.BlockSpec((1, bs, d), lambda bi, si: (bi, si, 0))
    mat_dr = pl.BlockSpec((1, d, r), lambda bi, si: (bi, 0, 0))
    mat_rd = pl.BlockSpec((1, r, d), lambda bi, si: (bi, 0, 0))
    out = jax.ShapeDtypeStruct((bsz, ssz, d), _BF16)
    return pl.pallas_call(
        functools.partial(_qkv_body, qscale=qscale),
        grid=(bsz, ssz // bs),
        in_specs=[big, mat_dr, mat_rd, mat_rd, mat_rd],
        out_specs=[big, big, big],
        out_shape=[out, out, out],
    )(h1, shared_c, sq, sk, sv)


def _attn_body(q_ref, k_ref, v_ref, x_ref, wo_ref, out_ref, *, scale,
               n_heads):
    q = q_ref[0]
    k = k_ref[0]
    v = v_ref[0]
    dh = q.shape[-1] // n_heads
    outs = []
    for hh in range(n_heads):
        sl = slice(hh * dh, (hh + 1) * dh)
        p = jnp.exp2(_dot_nt(q[:, sl], k[:, sl]))
        inv = 1.0 / jnp.sum(p, axis=-1, keepdims=True)
        outs.append(_dot(p.astype(jnp.bfloat16), v[:, sl]) * inv)
    o = jnp.concatenate(outs, axis=-1).astype(jnp.bfloat16)
    out_ref[0] = x_ref[0] + _dot(o, wo_ref[...])


def _attn(q, k, v, x, w_o, bq, scale, n_heads):
    bsz, ssz, d = q.shape
    blk = pl.BlockSpec((1, bq, d), lambda bi, qi: (bi, qi, 0))
    full = pl.BlockSpec((1, ssz, d), lambda bi, qi: (bi, 0, 0))
    return pl.pallas_call(
        functools.partial(_attn_body, scale=scale, n_heads=n_heads),
        grid=(bsz, ssz // bq),
        in_specs=[blk, full, full, blk,
                  pl.BlockSpec((d, d), lambda bi, qi: (0, 0))],
        out_specs=blk,
        out_shape=jax.ShapeDtypeStruct((bsz, ssz, d), _F32),
    )(q, k, v, x, w_o)


def _mem_body(h2_ref, sm_ref, kk_ref, kv_ref, x1_ref, out_ref, *, scale):
    hm = (_dot(h2_ref[0], sm_ref[0]) * scale).astype(jnp.bfloat16)
    p = jnp.exp2(_dot_nt(hm, kk_ref[...]))
    inv = 1.0 / jnp.sum(p, axis=-1, keepdims=True)
    out_ref[0] = x1_ref[0] + _dot(p.astype(jnp.bfloat16), kv_ref[...]) * inv


def _mem(h2, shared_m, k_k, k_v, x1, bs, scale):
    bsz, ssz, d = h2.shape
    r = shared_m.shape[-1]
    nk = k_k.shape[0]
    big = pl.BlockSpec((1, bs, d), lambda bi, si: (bi, si, 0))
    return pl.pallas_call(
        functools.partial(_mem_body, scale=scale),
        grid=(bsz, ssz // bs),
        in_specs=[
            big,
            pl.BlockSpec((1, d, r), lambda bi, si: (bi, 0, 0)),
            pl.BlockSpec((nk, r), lambda bi, si: (0, 0)),
            pl.BlockSpec((nk, d), lambda bi, si: (0, 0)),
            big,
        ],
        out_specs=big,
        out_shape=jax.ShapeDtypeStruct((bsz, ssz, d), _F32),
    )(h2, shared_m, k_k, k_v, x1)


def kernel(x, importance, W_compress_router, W_expand_router_Q,
           W_expand_router_K, W_expand_router_V, W_memory_router,
           compress_neurons, expand_neurons_pool, knowledge_K, knowledge_V,
           W_O, g1, b1, g2, b2):
    bsz, ssz, d = x.shape
    n_c, _, r = compress_neurons.shape
    nk = knowledge_K.shape[0]
    h = _N_HEADS
    dh = d // h
    bs = min(512, ssz)

    w_cat = jnp.concatenate([W_compress_router, W_expand_router_Q,
                             W_expand_router_K, W_expand_router_V], axis=1)
    h1, wacc, isum = _ln_route(x, importance, w_cat, g1, b1, 4, bs)
    tw, ix = _topk(wacc, isum, (16, 8, 8, 8))

    c_idx = ix[:, :16].reshape(bsz * 16)
    c_tw = tw[:, :16].reshape(bsz * 16)
    e_idx = ix[:, 16:].reshape(bsz, 3, 16)[:, :, :8].transpose(
        1, 0, 2).reshape(3 * bsz * 8)
    e_tw = tw[:, 16:].reshape(bsz, 3, 16)[:, :, :8].transpose(
        1, 0, 2).reshape(3 * bsz * 8)
    shared_c = _gcomb(c_idx, c_tw, compress_neurons, bsz, 16, d)
    shared_e = _gcomb(e_idx, e_tw, expand_neurons_pool, 3 * bsz, 8,
                      r).reshape(3, bsz, r, d)

    _LOG2E = 1.4426950408889634
    se16 = shared_e.astype(_BF16)
    q, k, v = _qkv(h1, shared_c.astype(_BF16), se16[0], se16[1], se16[2], bs,
                   _LOG2E / (dh ** 0.5))
    x1 = _attn(q, k, v, x, W_O.astype(_BF16), bs, 1.0 / (dh ** 0.5), h)

    h2, wacc_m, isum_m = _ln_route(x1, importance, W_memory_router, g2, b2,
                                   1, bs)
    tw_m, ix_m = _topk(wacc_m, isum_m, (16,))
    shared_m = _gcomb(ix_m.reshape(bsz * 16), tw_m.reshape(bsz * 16),
                      compress_neurons, bsz, 16, d)
    mbs = min(256, ssz)
    return _mem(h2, shared_m.astype(_BF16), knowledge_K.astype(_BF16),
                knowledge_V.astype(_BF16), x1, mbs,
                1.4426950408889634 / (r ** 0.5))
